# baseline (speedup 1.0000x reference)
import functools

import jax
import jax.numpy as jnp
from jax import lax
from jax.experimental import pallas as pl
from jax.experimental.pallas import tpu as pltpu
from jax.experimental.pallas import tpu_sc as plsc

_NW = 32
_C = 250


def _tc_prep(x3, W, b2, time_table, space_pair, nan_table):
    B, T, JD = x3.shape
    D = W.shape[1]
    half = JD // 2

    def body(x_ref, w_ref, b_ref, t_ref, sp_ref, n_ref, tbl_ref, idx_ref):
        xb = x_ref[...]
        mask = jnp.isnan(xb)
        xc = jnp.where(mask, jnp.float32(0.0), xb)
        lin = lax.dot_general(
            xc.reshape(B * T, JD), w_ref[...],
            (((1,), (0,)), ((), ())),
            preferred_element_type=jnp.float32,
        ).reshape(B, T, D)
        base = lin + (b_ref[0] + n_ref[0])[None, None, :] + t_ref[...][None, :, :]
        delta = (n_ref[1] - n_ref[0])[None, None, :]
        r0 = base + sp_ref[:, 0, :][None, :, :]
        r1 = base + sp_ref[:, 1, :][None, :, :]
        tbl_ref[:, :, 0, :] = r0
        tbl_ref[:, :, 1, :] = r0 + delta
        tbl_ref[:, :, 2, :] = r1
        tbl_ref[:, :, 3, :] = r1 + delta
        pair = (
            lax.broadcasted_iota(jnp.int32, (B, T, JD), 0) * T
            + lax.broadcasted_iota(jnp.int32, (B, T, JD), 1)
        )
        s_pos = lax.broadcasted_iota(jnp.int32, (B, T, JD), 2)
        idx_ref[...] = (
            pair * 4
            + jnp.where(s_pos >= half, jnp.int32(2), jnp.int32(0))
            + mask.astype(jnp.int32)
        )

    return pl.pallas_call(
        body,
        out_shape=(
            jax.ShapeDtypeStruct((B, T, 4, D), jnp.float32),
            jax.ShapeDtypeStruct((B, T, JD), jnp.int32),
        ),
    )(x3, W, b2, time_table, space_pair, nan_table)


def _sc_gather(tbl, idx2, n_rows, d):
    rows_per_w = n_rows // _NW
    nchunks = rows_per_w // _C
    mesh = plsc.VectorSubcoreMesh(core_axis_name="c", subcore_axis_name="s")

    @functools.partial(
        pl.kernel,
        out_type=jax.ShapeDtypeStruct((n_rows, d), jnp.float32),
        mesh=mesh,
        compiler_params=pltpu.CompilerParams(use_tc_tiling_on_sc=False),
        scratch_types=[
            pltpu.VMEM((nchunks, _C), jnp.int32),
            pltpu.VMEM((2, _C, d), jnp.float32),
            pltpu.SemaphoreType.DMA,
            pltpu.SemaphoreType.DMA,
            pltpu.SemaphoreType.DMA,
            pltpu.SemaphoreType.DMA,
        ],
    )
    def k(tbl_hbm, idx_hbm, out_hbm, idx_v, rows_v, gs0, gs1, os0, os1):
        wid = lax.axis_index("s") * 2 + lax.axis_index("c")
        base = wid * rows_per_w
        pltpu.sync_copy(idx_hbm.at[pl.ds(wid * nchunks, nchunks)], idx_v)

        def gfire(c, slot, sem):
            pltpu.async_copy(tbl_hbm.at[idx_v.at[c]], rows_v.at[slot], sem)

        def gwait(slot, sem):
            pltpu.make_async_copy(
                tbl_hbm.at[pl.ds(0, _C)], rows_v.at[slot], sem
            ).wait()

        def ofire(c, slot, sem):
            pltpu.async_copy(
                rows_v.at[slot], out_hbm.at[pl.ds(base + c * _C, _C)], sem
            )

        def owait(slot, sem):
            pltpu.make_async_copy(
                out_hbm.at[pl.ds(0, _C)], rows_v.at[slot], sem
            ).wait()

        gfire(0, 0, gs0)
        gfire(1, 1, gs1)

        def ring(j, carry):
            c0 = 2 * j
            gwait(0, gs0)
            ofire(c0, 0, os0)

            @pl.when(c0 + 2 < nchunks)
            def _():
                owait(0, os0)
                gfire(c0 + 2, 0, gs0)

            gwait(1, gs1)
            ofire(c0 + 1, 1, os1)

            @pl.when(c0 + 3 < nchunks)
            def _():
                owait(1, os1)
                gfire(c0 + 3, 1, gs1)

            return carry

        lax.fori_loop(0, nchunks // 2, ring, 0)
        owait(0, os0)
        owait(1, os1)

    return k(tbl, idx2)


def kernel(x, W, b, time_table, space_table, nan_table):
    B, T, J, DX = x.shape
    JD = J * DX
    D = W.shape[1]
    x3 = x.reshape(B, T, JD)
    space_pair = space_table.reshape(T, 2, D)
    tbl, idx = _tc_prep(
        x3, W, b.reshape(1, D), time_table, space_pair, nan_table
    )
    n_rows = B * T * JD
    out = _sc_gather(
        tbl.reshape(B * T * 4, D),
        idx.reshape(n_rows // _C, _C),
        n_rows,
        D,
    )
    return out.reshape(B, T * JD, D)


# speedup vs baseline: 1.3497x; 1.3497x over previous
import functools

import jax
import jax.numpy as jnp
from jax import lax
from jax.experimental import pallas as pl
from jax.experimental.pallas import tpu as pltpu
from jax.experimental.pallas import tpu_sc as plsc

_NW = 32
_LANES = 16
_P_SC = 1536
_PP = 128


def _tc_prep(x3, W, b2, time_table, space_pair, nan_table):
    B, T, JD = x3.shape
    D = W.shape[1]

    JDP = ((JD + _LANES - 1) // _LANES) * _LANES

    def body(x_ref, w_ref, b_ref, t_ref, sp_ref, n_ref, r0_ref, r1_ref, m_ref):
        xb = x_ref[...]
        mask = jnp.isnan(xb)
        xc = jnp.where(mask, jnp.float32(0.0), xb)
        lin = lax.dot_general(
            xc.reshape(B * T, JD), w_ref[...],
            (((1,), (0,)), ((), ())),
            preferred_element_type=jnp.float32,
        ).reshape(B, T, D)
        base = lin + (b_ref[0] + n_ref[0])[None, None, :] + t_ref[...][None, :, :]
        r0_ref[...] = base + sp_ref[:, 0, :][None, :, :]
        r1_ref[...] = base + sp_ref[:, 1, :][None, :, :]
        mf = mask.astype(jnp.float32)
        m_ref[...] = jnp.concatenate(
            [mf, jnp.zeros((B, T, JDP - JD), jnp.float32)], axis=2
        )

    return pl.pallas_call(
        body,
        out_shape=(
            jax.ShapeDtypeStruct((B, T, D), jnp.float32),
            jax.ShapeDtypeStruct((B, T, D), jnp.float32),
            jax.ShapeDtypeStruct((B, T, JDP), jnp.float32),
        ),
    )(x3, W, b2, time_table, space_pair, nan_table)


def _sc_expand(row0, row1, maskf, nan_table, n_pairs, jd, d):
    pairs_per_w = n_pairs // _NW
    half = jd // 2
    ncol = d // _LANES
    jdp = maskf.shape[1]
    nchunk = jdp // _LANES
    mesh = plsc.VectorSubcoreMesh(core_axis_name="c", subcore_axis_name="s")

    @functools.partial(
        pl.kernel,
        out_type=jax.ShapeDtypeStruct((n_pairs * jd, d), jnp.float32),
        mesh=mesh,
        compiler_params=pltpu.CompilerParams(use_tc_tiling_on_sc=False),
        scratch_types=[
            pltpu.VMEM((pairs_per_w, d), jnp.float32),
            pltpu.VMEM((pairs_per_w, d), jnp.float32),
            pltpu.VMEM((pairs_per_w, jdp), jnp.float32),
            pltpu.VMEM((2, d), jnp.float32),
            pltpu.VMEM((2, jd, d), jnp.float32),
            pltpu.SemaphoreType.DMA,
            pltpu.SemaphoreType.DMA,
        ],
    )
    def k(r0_hbm, r1_hbm, m_hbm, n_hbm, out_hbm, r0v, r1v, mv, nv, ov, sem0, sem1):
        wid = lax.axis_index("s") * 2 + lax.axis_index("c")
        base = wid * pairs_per_w
        pltpu.sync_copy(r0_hbm.at[pl.ds(base, pairs_per_w)], r0v)
        pltpu.sync_copy(r1_hbm.at[pl.ds(base, pairs_per_w)], r1v)
        pltpu.sync_copy(m_hbm.at[pl.ds(base, pairs_per_w)], mv)
        pltpu.sync_copy(n_hbm, nv)
        delta = [
            nv[1, pl.ds(_LANES * j, _LANES)] - nv[0, pl.ds(_LANES * j, _LANES)]
            for j in range(ncol)
        ]

        def build(p, buf):
            r0 = [r0v[p, pl.ds(_LANES * j, _LANES)] for j in range(ncol)]
            r1 = [r1v[p, pl.ds(_LANES * j, _LANES)] for j in range(ncol)]
            mvecs = [mv[p, pl.ds(_LANES * k, _LANES)] for k in range(nchunk)]
            for s in range(jd):
                src = r0 if s < half else r1
                m = mvecs[s // _LANES][s % _LANES]
                for j in range(ncol):
                    ov[buf, s, pl.ds(_LANES * j, _LANES)] = src[j] + m * delta[j]

        def drain(buf, sem):
            pltpu.make_async_copy(
                out_hbm.at[pl.ds(0, jd)], ov.at[buf], sem
            ).wait()

        def pair2_body(i, carry):
            p0 = 2 * i

            @pl.when(i > 0)
            def _():
                drain(0, sem0)

            build(p0, 0)
            pltpu.async_copy(ov.at[0], out_hbm.at[pl.ds((base + p0) * jd, jd)], sem0)

            @pl.when(i > 0)
            def _():
                drain(1, sem1)

            build(p0 + 1, 1)
            pltpu.async_copy(
                ov.at[1], out_hbm.at[pl.ds((base + p0 + 1) * jd, jd)], sem1
            )
            return carry

        lax.fori_loop(0, pairs_per_w // 2, pair2_body, 0)
        drain(0, sem0)
        drain(1, sem1)

    return k(row0, row1, maskf, nan_table)


def _tc_expand(row0, row1, maskT, nan_table, p_lo, n_pairs, jd, d):
    half = jd // 2
    nsteps = n_pairs // _PP
    off = p_lo // _PP

    def body(r0_ref, r1_ref, mT_ref, n_ref, o_ref):
        delta = (n_ref[1] - n_ref[0])[None, :]
        for q in range(_PP):
            r0 = r0_ref[q][None, :]
            r1 = r1_ref[q][None, :]
            mcol = mT_ref[:, q][:, None]
            rows = jnp.concatenate(
                [
                    jnp.broadcast_to(r0, (half, d)),
                    jnp.broadcast_to(r1, (half, d)),
                ],
                axis=0,
            )
            o_ref[q] = rows + mcol * delta

    out3 = pl.pallas_call(
        body,
        grid=(nsteps,),
        in_specs=[
            pl.BlockSpec((_PP, d), lambda i: (i + off, 0)),
            pl.BlockSpec((_PP, d), lambda i: (i + off, 0)),
            pl.BlockSpec((jd, _PP), lambda i: (0, i + off)),
            pl.BlockSpec((2, d), lambda i: (0, 0)),
        ],
        out_specs=pl.BlockSpec((_PP, jd, d), lambda i: (i, 0, 0)),
        out_shape=jax.ShapeDtypeStruct((n_pairs, jd, d), jnp.float32),
    )(row0, row1, maskT, nan_table)
    return out3.reshape(n_pairs * jd, d)


def kernel(x, W, b, time_table, space_table, nan_table):
    B, T, J, DX = x.shape
    JD = J * DX
    D = W.shape[1]
    x3 = x.reshape(B, T, JD)
    space_pair = space_table.reshape(T, 2, D)
    row0, row1, maskf = _tc_prep(
        x3, W, b.reshape(1, D), time_table, space_pair, nan_table
    )
    n_pairs = B * T
    r0f = row0.reshape(n_pairs, D)
    r1f = row1.reshape(n_pairs, D)
    mf = maskf.reshape(n_pairs, maskf.shape[2])
    sc_out = _sc_expand(r0f, r1f, mf, nan_table, _P_SC, JD, D)
    maskT = mf[:, :JD].T
    tc_out = _tc_expand(
        r0f, r1f, maskT, nan_table, _P_SC, n_pairs - _P_SC, JD, D
    )
    out = jnp.concatenate([sc_out, tc_out], axis=0)
    return out.reshape(B, T * JD, D)


# speedup vs baseline: 1.7814x; 1.3199x over previous
import functools

import jax
import jax.numpy as jnp
from jax import lax
from jax.experimental import pallas as pl
from jax.experimental.pallas import tpu as pltpu
from jax.experimental.pallas import tpu_sc as plsc

_NW = 32
_LANES = 16
_GP = 20
_C = 200
_SEG = 8


def _tc_prep(x3, W, b2, time_table, space_pair, nan_table):
    B, T, JD = x3.shape
    D = W.shape[1]
    half = JD // 2
    JDP = ((JD + _LANES - 1) // _LANES) * _LANES

    def body(x_ref, w_ref, b_ref, t_ref, sp_ref, n_ref,
             tbl_ref, idx_ref, r0_ref, r1_ref, m_ref):
        xb = x_ref[...]
        mask = jnp.isnan(xb)
        xc = jnp.where(mask, jnp.float32(0.0), xb)
        lin = lax.dot_general(
            xc.reshape(B * T, JD), w_ref[...],
            (((1,), (0,)), ((), ())),
            preferred_element_type=jnp.float32,
        ).reshape(B, T, D)
        base = lin + (b_ref[0] + n_ref[0])[None, None, :] + t_ref[...][None, :, :]
        delta = (n_ref[1] - n_ref[0])[None, None, :]
        r0 = base + sp_ref[:, 0, :][None, :, :]
        r1 = base + sp_ref[:, 1, :][None, :, :]
        r0_ref[...] = r0
        r1_ref[...] = r1
        tbl_ref[:, :, 0, :] = r0
        tbl_ref[:, :, 1, :] = r0 + delta
        tbl_ref[:, :, 2, :] = r1
        tbl_ref[:, :, 3, :] = r1 + delta
        mf = mask.astype(jnp.float32)
        m_ref[...] = jnp.concatenate(
            [mf, jnp.zeros((B, T, JDP - JD), jnp.float32)], axis=2
        )
        pair = (
            lax.broadcasted_iota(jnp.int32, (B, T, JD), 0) * T
            + lax.broadcasted_iota(jnp.int32, (B, T, JD), 1)
        )
        s_pos = lax.broadcasted_iota(jnp.int32, (B, T, JD), 2)
        idx_ref[...] = (
            pair * 4
            + jnp.where(s_pos >= half, jnp.int32(2), jnp.int32(0))
            + mask.astype(jnp.int32)
        )

    return pl.pallas_call(
        body,
        out_shape=(
            jax.ShapeDtypeStruct((B, T, 4, D), jnp.float32),
            jax.ShapeDtypeStruct((B, T, JD), jnp.int32),
            jax.ShapeDtypeStruct((B, T, D), jnp.float32),
            jax.ShapeDtypeStruct((B, T, D), jnp.float32),
            jax.ShapeDtypeStruct((B, T, JDP), jnp.float32),
        ),
    )(x3, W, b2, time_table, space_pair, nan_table)


def _sc_expand(tbl, idx2, row0, row1, maskf, nan_table, n_pairs, jd, d):
    pairs_per_w = n_pairs // _NW
    vp = pairs_per_w - _GP
    nchunks = _GP * jd // _C
    idx_rows_w = _GP * jd // idx2.shape[1]
    half = jd // 2
    ncol = d // _LANES
    jdp = maskf.shape[1]
    nchunk = jdp // _LANES
    mesh = plsc.VectorSubcoreMesh(core_axis_name="c", subcore_axis_name="s")

    @functools.partial(
        pl.kernel,
        out_type=jax.ShapeDtypeStruct((n_pairs * jd, d), jnp.float32),
        mesh=mesh,
        compiler_params=pltpu.CompilerParams(use_tc_tiling_on_sc=False),
        scratch_types=[
            pltpu.VMEM((idx_rows_w, _C), jnp.int32),
            pltpu.VMEM((2, _C, d), jnp.float32),
            pltpu.VMEM((vp, d), jnp.float32),
            pltpu.VMEM((vp, d), jnp.float32),
            pltpu.VMEM((vp, jdp), jnp.float32),
            pltpu.VMEM((2, d), jnp.float32),
            pltpu.VMEM((2, jd, d), jnp.float32),
            pltpu.SemaphoreType.DMA,
            pltpu.SemaphoreType.DMA,
            pltpu.SemaphoreType.DMA,
            pltpu.SemaphoreType.DMA,
            pltpu.SemaphoreType.DMA,
            pltpu.SemaphoreType.DMA,
            pltpu.SemaphoreType.DMA,
        ],
    )
    def k(tbl_hbm, idx_hbm, r0_hbm, r1_hbm, m_hbm, n_hbm, out_hbm,
          idx_v, rows_v, r0v, r1v, mv, nv, ov,
          gs0, gs1, os0, os1, sem0, sem1, si):
        wid = lax.axis_index("s") * 2 + lax.axis_index("c")
        base = wid * pairs_per_w
        gbase = base * jd
        vbase = base + _GP

        def gfire(c, slot, sem):
            pltpu.async_copy(tbl_hbm.at[idx_v.at[c]], rows_v.at[slot], sem)

        def gwait(slot, sem):
            pltpu.make_async_copy(
                tbl_hbm.at[pl.ds(0, _C)], rows_v.at[slot], sem
            ).wait()

        def ofire(c, slot, sem):
            pltpu.async_copy(
                rows_v.at[slot], out_hbm.at[pl.ds(gbase + c * _C, _C)], sem
            )

        def owait(slot, sem):
            pltpu.make_async_copy(
                out_hbm.at[pl.ds(0, _C)], rows_v.at[slot], sem
            ).wait()

        idx_stride = pairs_per_w * jd // _C
        pltpu.sync_copy(idx_hbm.at[pl.ds(wid * idx_stride, idx_rows_w)], idx_v)
        gfire(0, 0, gs0)
        gfire(1, 1, gs1)

        pltpu.async_copy(r0_hbm.at[pl.ds(vbase, vp)], r0v, si)
        pltpu.async_copy(r1_hbm.at[pl.ds(vbase, vp)], r1v, si)
        pltpu.async_copy(m_hbm.at[pl.ds(vbase, vp)], mv, si)
        pltpu.async_copy(n_hbm, nv, si)
        pltpu.make_async_copy(r0_hbm.at[pl.ds(0, vp)], r0v, si).wait()
        pltpu.make_async_copy(r1_hbm.at[pl.ds(0, vp)], r1v, si).wait()
        pltpu.make_async_copy(m_hbm.at[pl.ds(0, vp)], mv, si).wait()
        pltpu.make_async_copy(n_hbm, nv, si).wait()

        delta = [
            nv[1, pl.ds(_LANES * j, _LANES)] - nv[0, pl.ds(_LANES * j, _LANES)]
            for j in range(ncol)
        ]

        def build(p, buf):
            r0 = [r0v[p, pl.ds(_LANES * j, _LANES)] for j in range(ncol)]
            r1 = [r1v[p, pl.ds(_LANES * j, _LANES)] for j in range(ncol)]
            mvecs = [mv[p, pl.ds(_LANES * k, _LANES)] for k in range(nchunk)]
            for s in range(jd):
                src = r0 if s < half else r1
                m = mvecs[s // _LANES][s % _LANES]
                for j in range(ncol):
                    ov[buf, s, pl.ds(_LANES * j, _LANES)] = src[j] + m * delta[j]

        def vdrain(buf, sem):
            pltpu.make_async_copy(
                out_hbm.at[pl.ds(0, jd)], ov.at[buf], sem
            ).wait()

        def build2(i, carry):
            p0 = 2 * i

            @pl.when(i > 0)
            def _():
                vdrain(0, sem0)

            build(p0, 0)
            pltpu.async_copy(
                ov.at[0], out_hbm.at[pl.ds((vbase + p0) * jd, jd)], sem0
            )

            @pl.when(i > 0)
            def _():
                vdrain(1, sem1)

            build(p0 + 1, 1)
            pltpu.async_copy(
                ov.at[1], out_hbm.at[pl.ds((vbase + p0 + 1) * jd, jd)], sem1
            )
            return carry

        def ring2(j, carry):
            c0 = 2 * j
            gwait(0, gs0)
            ofire(c0, 0, os0)

            @pl.when(c0 + 2 < nchunks)
            def _():
                owait(0, os0)
                gfire(c0 + 2, 0, gs0)

            lax.fori_loop(j * _SEG, j * _SEG + _SEG // 2, build2, 0)

            gwait(1, gs1)
            ofire(c0 + 1, 1, os1)

            @pl.when(c0 + 3 < nchunks)
            def _():
                owait(1, os1)
                gfire(c0 + 3, 1, gs1)

            lax.fori_loop(
                j * _SEG + _SEG // 2, (j + 1) * _SEG // 2 * 2, build2, 0
            )
            return carry

        lax.fori_loop(0, nchunks // 2, ring2, 0)
        owait(0, os0)
        owait(1, os1)
        vdrain(0, sem0)
        vdrain(1, sem1)

    return k(tbl, idx2, row0, row1, maskf, nan_table)


def kernel(x, W, b, time_table, space_table, nan_table):
    B, T, J, DX = x.shape
    JD = J * DX
    D = W.shape[1]
    x3 = x.reshape(B, T, JD)
    space_pair = space_table.reshape(T, 2, D)
    tbl, idx, row0, row1, maskf = _tc_prep(
        x3, W, b.reshape(1, D), time_table, space_pair, nan_table
    )
    n_pairs = B * T
    out = _sc_expand(
        tbl.reshape(n_pairs * 4, D),
        idx.reshape(n_pairs * JD // _C, _C),
        row0.reshape(n_pairs, D),
        row1.reshape(n_pairs, D),
        maskf.reshape(n_pairs, maskf.shape[2]),
        nan_table,
        n_pairs,
        JD,
        D,
    )
    return out.reshape(B, T * JD, D)


# speedup vs baseline: 2.6531x; 1.4893x over previous
import functools

import jax
import jax.numpy as jnp
from jax import lax
from jax.experimental import pallas as pl
from jax.experimental.pallas import tpu as pltpu
from jax.experimental.pallas import tpu_sc as plsc

_NW = 32
_LANES = 16


def _tc_prep(x3, W, b2, time_table, space_pair, nan_table):
    B, T, JD = x3.shape
    D = W.shape[1]

    JDP = ((JD + _LANES - 1) // _LANES) * _LANES

    def body(x_ref, w_ref, b_ref, t_ref, sp_ref, n_ref, r0_ref, r1_ref, m_ref):
        xb = x_ref[...]
        mask = jnp.isnan(xb)
        xc = jnp.where(mask, jnp.float32(0.0), xb)
        lin = lax.dot_general(
            xc.reshape(B * T, JD), w_ref[...],
            (((1,), (0,)), ((), ())),
            preferred_element_type=jnp.float32,
        ).reshape(B, T, D)
        base = lin + (b_ref[0] + n_ref[0])[None, None, :] + t_ref[...][None, :, :]
        r0_ref[...] = base + sp_ref[:, 0, :][None, :, :]
        r1_ref[...] = base + sp_ref[:, 1, :][None, :, :]
        mf = mask.astype(jnp.float32)
        m_ref[...] = jnp.concatenate(
            [mf, jnp.zeros((B, T, JDP - JD), jnp.float32)], axis=2
        )

    return pl.pallas_call(
        body,
        out_shape=(
            jax.ShapeDtypeStruct((B, T, D), jnp.float32),
            jax.ShapeDtypeStruct((B, T, D), jnp.float32),
            jax.ShapeDtypeStruct((B, T, JDP), jnp.float32),
        ),
    )(x3, W, b2, time_table, space_pair, nan_table)


def _sc_expand(row0, row1, maskf, nan_table, n_pairs, jd, d):
    pairs_per_w = n_pairs // _NW
    half = jd // 2
    ncol = d // _LANES
    jdp = maskf.shape[1]
    nchunk = jdp // _LANES
    mesh = plsc.VectorSubcoreMesh(core_axis_name="c", subcore_axis_name="s")

    @functools.partial(
        pl.kernel,
        out_type=jax.ShapeDtypeStruct((n_pairs * jd, d), jnp.float32),
        mesh=mesh,
        compiler_params=pltpu.CompilerParams(use_tc_tiling_on_sc=False),
        scratch_types=[
            pltpu.VMEM((pairs_per_w, d), jnp.float32),
            pltpu.VMEM((pairs_per_w, d), jnp.float32),
            pltpu.VMEM((pairs_per_w, jdp), jnp.float32),
            pltpu.VMEM((2, d), jnp.float32),
            pltpu.VMEM((jd, d), jnp.float32),
        ],
    )
    def k(r0_hbm, r1_hbm, m_hbm, n_hbm, out_hbm, r0v, r1v, mv, nv, ov):
        wid = lax.axis_index("s") * 2 + lax.axis_index("c")
        base = wid * pairs_per_w
        pltpu.sync_copy(r0_hbm.at[pl.ds(base, pairs_per_w)], r0v)
        pltpu.sync_copy(r1_hbm.at[pl.ds(base, pairs_per_w)], r1v)
        pltpu.sync_copy(m_hbm.at[pl.ds(base, pairs_per_w)], mv)
        pltpu.sync_copy(n_hbm, nv)
        delta = [
            nv[1, pl.ds(_LANES * j, _LANES)] - nv[0, pl.ds(_LANES * j, _LANES)]
            for j in range(ncol)
        ]

        def pair_body(p, carry):
            r0 = [r0v[p, pl.ds(_LANES * j, _LANES)] for j in range(ncol)]
            r1 = [r1v[p, pl.ds(_LANES * j, _LANES)] for j in range(ncol)]
            mvecs = [mv[p, pl.ds(_LANES * k, _LANES)] for k in range(nchunk)]
            for s in range(jd):
                src = r0 if s < half else r1
                m = mvecs[s // _LANES][s % _LANES]
                for j in range(ncol):
                    ov[s, pl.ds(_LANES * j, _LANES)] = src[j] + m * delta[j]
            pltpu.sync_copy(ov, out_hbm.at[pl.ds((base + p) * jd, jd)])
            return carry

        lax.fori_loop(0, pairs_per_w, pair_body, 0)

    return k(row0, row1, maskf, nan_table)


def kernel(x, W, b, time_table, space_table, nan_table):
    B, T, J, DX = x.shape
    JD = J * DX
    D = W.shape[1]
    x3 = x.reshape(B, T, JD)
    space_pair = space_table.reshape(T, 2, D)
    row0, row1, maskf = _tc_prep(
        x3, W, b.reshape(1, D), time_table, space_pair, nan_table
    )
    out = _sc_expand(
        row0.reshape(B * T, D),
        row1.reshape(B * T, D),
        maskf.reshape(B * T, maskf.shape[2]),
        nan_table,
        B * T,
        JD,
        D,
    )
    return out.reshape(B, T * JD, D)


# speedup vs baseline: 2.7009x; 1.0180x over previous
import functools

import jax
import jax.numpy as jnp
from jax import lax
from jax.experimental import pallas as pl
from jax.experimental.pallas import tpu as pltpu
from jax.experimental.pallas import tpu_sc as plsc

_NW = 32
_LANES = 16


def _tc_prep(x3, W, b2, time_table, space_pair, nan_table):
    B, T, JD = x3.shape
    D = W.shape[1]

    JDP = ((JD + _LANES - 1) // _LANES) * _LANES

    def body(x_ref, w_ref, b_ref, t_ref, sp_ref, n_ref, r0_ref, r1_ref, m_ref):
        xb = x_ref[...]
        mask = jnp.isnan(xb)
        xc = jnp.where(mask, jnp.float32(0.0), xb)
        lin = lax.dot_general(
            xc.reshape(B * T, JD), w_ref[...],
            (((1,), (0,)), ((), ())),
            preferred_element_type=jnp.float32,
        ).reshape(B, T, D)
        base = lin + (b_ref[0] + n_ref[0])[None, None, :] + t_ref[...][None, :, :]
        r0_ref[...] = base + sp_ref[:, 0, :][None, :, :]
        r1_ref[...] = base + sp_ref[:, 1, :][None, :, :]
        mf = mask.astype(jnp.float32)
        m_ref[...] = jnp.concatenate(
            [mf, jnp.zeros((B, T, JDP - JD), jnp.float32)], axis=2
        )

    return pl.pallas_call(
        body,
        out_shape=(
            jax.ShapeDtypeStruct((B, T, D), jnp.float32),
            jax.ShapeDtypeStruct((B, T, D), jnp.float32),
            jax.ShapeDtypeStruct((B, T, JDP), jnp.float32),
        ),
    )(x3, W, b2, time_table, space_pair, nan_table)


def _sc_expand(row0, row1, maskf, nan_table, n_pairs, jd, d):
    pairs_per_w = n_pairs // _NW
    half = jd // 2
    ncol = d // _LANES
    jdp = maskf.shape[1]
    nchunk = jdp // _LANES
    mesh = plsc.VectorSubcoreMesh(core_axis_name="c", subcore_axis_name="s")

    @functools.partial(
        pl.kernel,
        out_type=jax.ShapeDtypeStruct((n_pairs * jd, d), jnp.float32),
        mesh=mesh,
        compiler_params=pltpu.CompilerParams(use_tc_tiling_on_sc=False),
        scratch_types=[
            pltpu.VMEM((pairs_per_w, d), jnp.float32),
            pltpu.VMEM((pairs_per_w, d), jnp.float32),
            pltpu.VMEM((pairs_per_w, jdp), jnp.float32),
            pltpu.VMEM((2, d), jnp.float32),
            pltpu.VMEM((jd, d), jnp.float32),
            pltpu.SemaphoreType.DMA,
        ],
    )
    def k(r0_hbm, r1_hbm, m_hbm, n_hbm, out_hbm, r0v, r1v, mv, nv, ov, si):
        wid = lax.axis_index("s") * 2 + lax.axis_index("c")
        base = wid * pairs_per_w
        pltpu.async_copy(r0_hbm.at[pl.ds(base, pairs_per_w)], r0v, si)
        pltpu.async_copy(r1_hbm.at[pl.ds(base, pairs_per_w)], r1v, si)
        pltpu.async_copy(m_hbm.at[pl.ds(base, pairs_per_w)], mv, si)
        pltpu.async_copy(n_hbm, nv, si)
        pltpu.make_async_copy(r0_hbm.at[pl.ds(0, pairs_per_w)], r0v, si).wait()
        pltpu.make_async_copy(r1_hbm.at[pl.ds(0, pairs_per_w)], r1v, si).wait()
        pltpu.make_async_copy(m_hbm.at[pl.ds(0, pairs_per_w)], mv, si).wait()
        pltpu.make_async_copy(n_hbm, nv, si).wait()
        delta = [
            nv[1, pl.ds(_LANES * j, _LANES)] - nv[0, pl.ds(_LANES * j, _LANES)]
            for j in range(ncol)
        ]

        def pair_body(p, carry):
            r0 = [r0v[p, pl.ds(_LANES * j, _LANES)] for j in range(ncol)]
            r1 = [r1v[p, pl.ds(_LANES * j, _LANES)] for j in range(ncol)]
            mvecs = [mv[p, pl.ds(_LANES * k, _LANES)] for k in range(nchunk)]
            for s in range(jd):
                src = r0 if s < half else r1
                m = mvecs[s // _LANES][s % _LANES]
                for j in range(ncol):
                    ov[s, pl.ds(_LANES * j, _LANES)] = src[j] + m * delta[j]
            pltpu.sync_copy(ov, out_hbm.at[pl.ds((base + p) * jd, jd)])
            return carry

        lax.fori_loop(0, pairs_per_w, pair_body, 0)

    return k(row0, row1, maskf, nan_table)


def kernel(x, W, b, time_table, space_table, nan_table):
    B, T, J, DX = x.shape
    JD = J * DX
    D = W.shape[1]
    x3 = x.reshape(B, T, JD)
    space_pair = space_table.reshape(T, 2, D)
    row0, row1, maskf = _tc_prep(
        x3, W, b.reshape(1, D), time_table, space_pair, nan_table
    )
    out = _sc_expand(
        row0.reshape(B * T, D),
        row1.reshape(B * T, D),
        maskf.reshape(B * T, maskf.shape[2]),
        nan_table,
        B * T,
        JD,
        D,
    )
    return out.reshape(B, T * JD, D)
